# baseline (speedup 1.0000x reference)
import functools

import jax
import jax.numpy as jnp
from jax import lax
from jax.experimental import pallas as pl
from jax.experimental.pallas import tpu as pltpu
from jax.experimental.pallas import tpu_sc as plsc

B = 16384
D = 32


def kernel(user_id, item_id, context_id, user_table, item_table, context_table):
    info = plsc.get_sparse_core_info()
    nc, ns = info.num_cores, info.num_subcores
    nw = nc * ns
    b_per_w = B // nw

    mesh = plsc.VectorSubcoreMesh(core_axis_name="c", subcore_axis_name="s")

    @functools.partial(
        pl.kernel,
        mesh=mesh,
        out_type=jax.ShapeDtypeStruct((B * D,), jnp.float32),
        compiler_params=pltpu.CompilerParams(use_tc_tiling_on_sc=False),
        scratch_types=[
            pltpu.VMEM((b_per_w,), jnp.int32),
            pltpu.VMEM((b_per_w,), jnp.int32),
            pltpu.VMEM((b_per_w,), jnp.int32),
            pltpu.VMEM((b_per_w, D), jnp.float32),
            pltpu.VMEM((b_per_w, D), jnp.float32),
            pltpu.VMEM((b_per_w, D), jnp.float32),
            pltpu.VMEM((b_per_w * D,), jnp.float32),
            pltpu.SemaphoreType.DMA,
            pltpu.SemaphoreType.DMA,
            pltpu.SemaphoreType.DMA,
        ],
    )
    def sc_kernel(uid_hbm, iid_hbm, cid_hbm, ut_hbm, it_hbm, ct_hbm, out_hbm,
                  uidx_v, iidx_v, cidx_v, urows, irows, crows, out_v,
                  sem_u, sem_i, sem_c):
        wid = lax.axis_index("s") * nc + lax.axis_index("c")
        base = wid * b_per_w
        pltpu.sync_copy(uid_hbm.at[pl.ds(base, b_per_w)], uidx_v)
        pltpu.sync_copy(iid_hbm.at[pl.ds(base, b_per_w)], iidx_v)
        pltpu.sync_copy(cid_hbm.at[pl.ds(base, b_per_w)], cidx_v)
        cu = pltpu.async_copy(ut_hbm.at[uidx_v], urows, sem_u)
        ci = pltpu.async_copy(it_hbm.at[iidx_v], irows, sem_i)
        cc = pltpu.async_copy(ct_hbm.at[cidx_v], crows, sem_c)
        cu.wait()
        ci.wait()
        cc.wait()

        third = jnp.float32(1.0 / 3.0)

        def avg_body(r, carry):
            for col in range(0, D, 16):
                s = pl.ds(col, 16)
                out_v[pl.ds(r * D + col, 16)] = (
                    urows[r, s] + irows[r, s] + crows[r, s]) * third
            return carry

        lax.fori_loop(0, b_per_w, avg_body, 0)
        pltpu.sync_copy(out_v, out_hbm.at[pl.ds(base * D, b_per_w * D)])

    out_flat = sc_kernel(user_id, item_id, context_id,
                         user_table, item_table, context_table)
    return out_flat.reshape(B, D)
